# speedup vs baseline: 11.5931x; 11.5931x over previous
import functools

import jax
import jax.numpy as jnp
from jax.experimental import pallas as pl
from jax.experimental.pallas import tpu as pltpu


def _rup(x, m):
    return ((x + m - 1) // m) * m


def _im2col_dot(xs_ref, cols_ref, w_ref, mask_ref, *, C, HWo, Wo, HP, Mp, full):
    t = 0
    for kd in range(3):
        for kh in range(3):
            for kw in range(3):
                off = HP + (kd - 1) * HWo + (kh - 1) * Wo + (kw - 1)
                tap = xs_ref[:, off:off + Mp]
                mi = 3 * kh + kw
                if not (full and mi == 4):
                    tap = tap * mask_ref[mi:mi + 1, :]
                cols_ref[t * C:(t + 1) * C, :] = tap
                t += 1
    return jnp.dot(w_ref[...], cols_ref[...], preferred_element_type=jnp.float32)


def _pool_conv1_kernel(x_ref, sel_ref, w_ref, mask_ref,
                       y_ref, ssum_ref, ssq_ref,
                       pool_ref, xs_ref, cols_ref, *,
                       C, Do, HW_in, W_in, HWo, Wo, HP, M, Mp, full):
    for do in range(Do):
        v = x_ref[0, :, 2 * HW_in * do:2 * HW_in * (do + 1)]
        a = jnp.maximum(v[:, :HW_in], v[:, HW_in:])
        b = jnp.maximum(a, jnp.concatenate([a[:, W_in:], a[:, :W_in]], axis=1))
        c = jnp.maximum(b, jnp.concatenate([b[:, 1:], b[:, :1]], axis=1))
        pool_ref[do * C:(do + 1) * C, :] = c.astype(jnp.bfloat16)
    p = jnp.dot(pool_ref[...], sel_ref[...],
                preferred_element_type=jnp.float32)
    p = p.astype(jnp.bfloat16)

    L = xs_ref.shape[1]
    xs_ref[:, :HP] = jnp.zeros((C, HP), jnp.bfloat16)
    for do in range(Do):
        xs_ref[:, HP + do * HWo:HP + (do + 1) * HWo] = p[do * C:(do + 1) * C, :]
    xs_ref[:, HP + M:] = jnp.zeros((C, L - HP - M), jnp.bfloat16)

    acc = _im2col_dot(xs_ref, cols_ref, w_ref, mask_ref,
                      C=C, HWo=HWo, Wo=Wo, HP=HP, Mp=Mp, full=full)
    cout = y_ref.shape[1]
    y_ref[0, :, :HP] = jnp.zeros((cout, HP), jnp.bfloat16)
    y_ref[0, :, HP:HP + Mp] = acc.astype(jnp.bfloat16)
    y_ref[0, :, HP + Mp:] = jnp.zeros((cout, L - HP - Mp), jnp.bfloat16)
    ssum_ref[0] = jnp.sum(acc, axis=1, keepdims=True)
    ssq_ref[0] = jnp.sum(acc * acc, axis=1, keepdims=True)


def _conv2_kernel(y1_ref, scale_ref, shift_ref, valid_ref, w_ref, mask_ref,
                  y2_ref, ssum_ref, ssq_ref,
                  xs_ref, cols_ref, *, C, HWo, Wo, HP, Mp, full):
    yv = y1_ref[0].astype(jnp.float32)
    act = jnp.maximum(yv * scale_ref[...] + shift_ref[...], 0.0)
    xs_ref[...] = (act * valid_ref[...]).astype(jnp.bfloat16)
    acc = _im2col_dot(xs_ref, cols_ref, w_ref, mask_ref,
                      C=C, HWo=HWo, Wo=Wo, HP=HP, Mp=Mp, full=full)
    y2_ref[0] = acc.astype(jnp.bfloat16)
    ssum_ref[0] = jnp.sum(acc, axis=1, keepdims=True)
    ssq_ref[0] = jnp.sum(acc * acc, axis=1, keepdims=True)


def _bn_relu_out_kernel(y_ref, scale_ref, shift_ref, o_ref):
    o_ref[0] = jnp.maximum(
        y_ref[0].astype(jnp.float32) * scale_ref[...] + shift_ref[...], 0.0)


def _fold_w(w):
    cout, cin = w.shape[0], w.shape[1]
    wt = jnp.transpose(w.astype(jnp.float32), (2, 3, 4, 0, 1))
    wt = jnp.transpose(wt.reshape(27, cout, cin), (1, 0, 2))
    return wt.reshape(cout, 27 * cin).astype(jnp.bfloat16)


def _fold_bn(ssum, ssq, count, gamma, beta, eps=1e-5):
    s = jnp.sum(ssum[:, :, 0], axis=0)
    sq = jnp.sum(ssq[:, :, 0], axis=0)
    mean = s / count
    var = sq / count - mean * mean
    inv = gamma / jnp.sqrt(var + eps)
    scale = inv.reshape(-1, 1).astype(jnp.float32)
    shift = (beta - mean * inv).reshape(-1, 1).astype(jnp.float32)
    return scale, shift


def kernel(x, w1, g1, be1, w2, g2, be2):
    N, Cin, D, H, W = x.shape
    C1, C2 = w1.shape[0], w2.shape[0]
    assert Cin % 8 == 0 and C1 % 8 == 0 and C2 % 8 == 0
    assert D % 2 == 0 and H % 2 == 0 and W % 2 == 0
    Do, Ho, Wo = D // 2, H // 2, W // 2
    HWo = Ho * Wo
    M = Do * HWo
    Mp = _rup(M, 128)
    HP = _rup(HWo + Wo + 1, 128)
    L = HP + Mp + HP
    HW_in = H * W
    full = (M == Mp)

    xr = x.reshape(N, Cin, D * HW_in)

    l_idx = jnp.arange(HW_in)[:, None]
    k_idx = jnp.arange(HWo)[None, :]
    sel = (l_idx == 2 * W * (k_idx // Wo) + 2 * (k_idx % Wo)).astype(jnp.bfloat16)
    m = jnp.arange(Mp)
    w_i = m % Wo
    h_i = (m // Wo) % Ho
    rows = []
    for kh in range(3):
        for kw in range(3):
            ok = ((h_i + kh - 1 >= 0) & (h_i + kh - 1 < Ho)
                  & (w_i + kw - 1 >= 0) & (w_i + kw - 1 < Wo) & (m < M))
            rows.append(ok)
    mask = jnp.stack(rows, axis=0).astype(jnp.bfloat16)
    lane = jnp.arange(L)
    valid = ((lane >= HP) & (lane < HP + M)).astype(jnp.float32).reshape(1, L)

    w1f = _fold_w(w1)
    w2f = _fold_w(w2)

    k1 = functools.partial(_pool_conv1_kernel, C=Cin, Do=Do, HW_in=HW_in,
                           W_in=W, HWo=HWo, Wo=Wo, HP=HP, M=M, Mp=Mp, full=full)
    y1, s1, q1 = pl.pallas_call(
        k1,
        out_shape=(jax.ShapeDtypeStruct((N, C1, L), jnp.bfloat16),
                   jax.ShapeDtypeStruct((N, C1, 1), jnp.float32),
                   jax.ShapeDtypeStruct((N, C1, 1), jnp.float32)),
        grid=(N,),
        in_specs=[pl.BlockSpec((1, Cin, D * HW_in), lambda n: (n, 0, 0)),
                  pl.BlockSpec((HW_in, HWo), lambda n: (0, 0)),
                  pl.BlockSpec((C1, 27 * Cin), lambda n: (0, 0)),
                  pl.BlockSpec((9, Mp), lambda n: (0, 0))],
        out_specs=(pl.BlockSpec((1, C1, L), lambda n: (n, 0, 0)),
                   pl.BlockSpec((1, C1, 1), lambda n: (n, 0, 0)),
                   pl.BlockSpec((1, C1, 1), lambda n: (n, 0, 0))),
        scratch_shapes=[pltpu.VMEM((Do * Cin, HW_in), jnp.bfloat16),
                        pltpu.VMEM((Cin, L), jnp.bfloat16),
                        pltpu.VMEM((27 * Cin, Mp), jnp.bfloat16)],
        compiler_params=pltpu.CompilerParams(
            dimension_semantics=("parallel",)),
    )(xr, sel, w1f, mask)
    sc1, sh1 = _fold_bn(s1, q1, N * M, g1, be1)

    k2 = functools.partial(_conv2_kernel, C=C1, HWo=HWo, Wo=Wo, HP=HP, Mp=Mp,
                           full=full)
    y2, s2, q2 = pl.pallas_call(
        k2,
        out_shape=(jax.ShapeDtypeStruct((N, C2, Mp), jnp.bfloat16),
                   jax.ShapeDtypeStruct((N, C2, 1), jnp.float32),
                   jax.ShapeDtypeStruct((N, C2, 1), jnp.float32)),
        grid=(N,),
        in_specs=[pl.BlockSpec((1, C1, L), lambda n: (n, 0, 0)),
                  pl.BlockSpec((C1, 1), lambda n: (0, 0)),
                  pl.BlockSpec((C1, 1), lambda n: (0, 0)),
                  pl.BlockSpec((1, L), lambda n: (0, 0)),
                  pl.BlockSpec((C2, 27 * C1), lambda n: (0, 0)),
                  pl.BlockSpec((9, Mp), lambda n: (0, 0))],
        out_specs=(pl.BlockSpec((1, C2, Mp), lambda n: (n, 0, 0)),
                   pl.BlockSpec((1, C2, 1), lambda n: (n, 0, 0)),
                   pl.BlockSpec((1, C2, 1), lambda n: (n, 0, 0))),
        scratch_shapes=[pltpu.VMEM((C1, L), jnp.bfloat16),
                        pltpu.VMEM((27 * C1, Mp), jnp.bfloat16)],
        compiler_params=pltpu.CompilerParams(
            dimension_semantics=("parallel",)),
    )(y1, sc1, sh1, valid, w2f, mask)
    sc2, sh2 = _fold_bn(s2, q2, N * M, g2, be2)

    out = pl.pallas_call(
        _bn_relu_out_kernel,
        out_shape=jax.ShapeDtypeStruct((N, C2, Mp), jnp.float32),
        grid=(N,),
        in_specs=[pl.BlockSpec((1, C2, Mp), lambda n: (n, 0, 0)),
                  pl.BlockSpec((C2, 1), lambda n: (0, 0)),
                  pl.BlockSpec((C2, 1), lambda n: (0, 0))],
        out_specs=pl.BlockSpec((1, C2, Mp), lambda n: (n, 0, 0)),
        compiler_params=pltpu.CompilerParams(
            dimension_semantics=("parallel",)),
    )(y2, sc2, sh2)
    return out[:, :, :M].reshape(N, C2, Do, Ho, Wo)
